# baseline (speedup 1.0000x reference)
import jax
import jax.numpy as jnp
from jax.experimental import pallas as pl
from jax.experimental.pallas import tpu as pltpu

RB = 512
XB = 1024
CHUNK = 2736
K = 8192
WIDTHS = (CHUNK, CHUNK, K - 2 * CHUNK)


def _bf16_round(v):
    return v.astype(jnp.bfloat16).astype(jnp.float32)


def _x2_kernel(x_ref, x2_ref):
    xs = x_ref[...] * -2.0
    x2 = 0.25 * jnp.sum(xs * xs, axis=1, keepdims=True)
    x2_ref[...] = jnp.broadcast_to(x2, (x2.shape[0], 128))


def _vq_kernel(x_ref, c_ref, x2_ref, out_ref, y2_scr):
    i = pl.program_id(0)

    @pl.when(i == 0)
    def _():
        c = c_ref[...]
        y2row = jnp.sum(c * c, axis=1)[None, :]
        for j, w in enumerate(WIDTHS):
            lo = j * CHUNK
            y2_scr[j:j + 1, 0:w] = y2row[:, lo:lo + w]

    xs = x_ref[...] * -2.0
    x2 = x2_ref[:, 0:1]

    lane = jax.lax.broadcasted_iota(jnp.int32, (1, 128), 1)
    lane_f = lane.astype(jnp.float32)
    inf = jnp.float32(jnp.inf)

    def chunk_dot(j):
        w = WIDTHS[j]
        cj = c_ref[j * CHUNK:j * CHUNK + w, :]
        return jax.lax.dot_general(
            xs, cj, (((1,), (1,)), ((), ())),
            preferred_element_type=jnp.float32)

    def cols(a, t):
        return a[:, t * 128:(t + 1) * 128]

    def chunk_reduce(xy, j):
        w = WIDTHS[j]
        nfull, tail = divmod(w, 128)
        y2j = y2_scr[j:j + 1, 0:w]
        d2 = (x2 + y2j) + xy

        acc = cols(d2, 0)
        for t in range(1, nfull):
            acc = jnp.minimum(acc, cols(d2, t))
        tcol = d2[:, nfull * 128:w]
        mraw = jnp.minimum(jnp.min(acc, axis=1), jnp.min(tcol, axis=1))

        m = jnp.maximum(mraw, 0.0)
        s = jnp.sqrt(m)
        g = s * 4097.0
        sh = g - (g - s)
        sl = s - sh
        p = s * s
        e = ((sh * sh - p) + 2.0 * (sh * sl)) + sl * sl
        sb = jax.lax.bitcast_convert_type(s, jnp.int32)
        h = jax.lax.bitcast_convert_type(
            sb & jnp.int32(0x7F800000), jnp.float32) * jnp.float32(2.0 ** -23)
        t2 = e + s * h
        mb = jax.lax.bitcast_convert_type(m, jnp.int32)
        npass = jnp.int32(0)
        for j_ulp in range(1, 4):
            u = jax.lax.bitcast_convert_type(mb + j_ulp, jnp.float32)
            npass = npass + ((u - p) <= t2).astype(jnp.int32)
        hi_b = jax.lax.bitcast_convert_type(mb + 1 + npass, jnp.float32)
        b = hi_b[:, None]
        tf = jnp.where(cols(d2, 0) < b, jnp.float32(0.0), inf)
        for t in range(1, nfull):
            cand = jnp.where(cols(d2, t) < b, jnp.float32(t), inf)
            tf = jnp.minimum(tf, cand)
        tft = jnp.where(tcol < b, jnp.float32(nfull), inf)
        tmin = jnp.minimum(jnp.min(tf, axis=1), jnp.min(tft, axis=1))
        lm = jnp.where(tf == tmin[:, None], lane_f, inf)
        lmt = jnp.where(tft == tmin[:, None], lane_f[:, 0:tail], inf)
        lmin = jnp.minimum(jnp.min(lm, axis=1), jnp.min(lmt, axis=1))
        idx = (jnp.float32(j * CHUNK) + tmin * 128.0) + lmin
        return s, idx

    xy1 = chunk_dot(0)
    xy2 = chunk_dot(1)
    s1, i1 = chunk_reduce(xy1, 0)
    xy3 = chunk_dot(2)
    s2, i2 = chunk_reduce(xy2, 1)
    s3, i3 = chunk_reduce(xy3, 2)

    r = _bf16_round(s1)
    idx = i1
    upd = s2 < r
    idx = jnp.where(upd, i2, idx)
    r = _bf16_round(jnp.where(upd, s2, r))
    upd = s3 < r
    idx = jnp.where(upd, i3, idx)
    out_ref[0, 0, :] = idx.astype(jnp.int32)


def kernel(latents, codebook):
    b, l, d = latents.shape
    n = b * l
    x = latents.reshape(n, d)

    x2 = pl.pallas_call(
        _x2_kernel,
        grid=(n // XB,),
        in_specs=[pl.BlockSpec((XB, d), lambda i: (i, 0))],
        out_specs=pl.BlockSpec((XB, 128), lambda i: (i, 0)),
        out_shape=jax.ShapeDtypeStruct((n, 128), jnp.float32),
    )(x)

    nblk = n // RB
    out = pl.pallas_call(
        _vq_kernel,
        grid=(nblk,),
        in_specs=[
            pl.BlockSpec((RB, d), lambda i: (i, 0)),
            pl.BlockSpec((K, d), lambda i: (0, 0)),
            pl.BlockSpec((RB, 128), lambda i: (i, 0)),
        ],
        out_specs=pl.BlockSpec((1, 1, RB), lambda i: (i, 0, 0)),
        out_shape=jax.ShapeDtypeStruct((nblk, 1, RB), jnp.int32),
        scratch_shapes=[pltpu.VMEM((3, CHUNK), jnp.float32)],
    )(x, codebook, x2)
    return out.reshape(b, l)


# speedup vs baseline: 1.0627x; 1.0627x over previous
import jax
import jax.numpy as jnp
from jax.experimental import pallas as pl
from jax.experimental.pallas import tpu as pltpu

RB = 512
XB = 1024
CHUNK = 2736
K = 8192


def _bf16_round(v):
    return v.astype(jnp.bfloat16).astype(jnp.float32)


def _x2_kernel(x_ref, x2_ref):
    xs = x_ref[...] * -2.0
    x2 = 0.25 * jnp.sum(xs * xs, axis=1, keepdims=True)
    x2_ref[...] = jnp.broadcast_to(x2, (x2.shape[0], 128))


def _vq_kernel(x_ref, c_ref, x2_ref, out_ref, y2_scr):
    i = pl.program_id(0)

    @pl.when(i == 0)
    def _():
        c = c_ref[...]
        y2_scr[...] = jnp.sum(c * c, axis=1)[None, :]

    xs = x_ref[...] * -2.0
    x2 = x2_ref[:, 0:1]
    y2 = y2_scr[...]

    half = K // 2

    def half_dot(j):
        cj = c_ref[j * half:(j + 1) * half, :]
        return jax.lax.dot_general(
            xs, cj, (((1,), (1,)), ((), ())),
            preferred_element_type=jnp.float32)

    xyA = half_dot(0)
    xyB = half_dot(1)
    d2A = (x2 + y2[:, 0:half]) + xyA
    d2B = (x2 + y2[:, half:K]) + xyB

    iota = jax.lax.broadcasted_iota(
        jnp.int32, (1, K), 1).astype(jnp.float32)
    lane = jax.lax.broadcasted_iota(jnp.int32, (1, 128), 1)
    inf = jnp.float32(jnp.inf)
    hvreg = half // 128

    def cols(a, t):
        if a is None:
            src, tt = (d2A, t) if t < hvreg else (d2B, t - hvreg)
            return src[:, tt * 128:(tt + 1) * 128]
        return a[:, t * 128:(t + 1) * 128]

    def chunk_min(d2, lo, hi):
        v0, r0 = divmod(lo, 128)
        v1, r1 = divmod(hi, 128)
        acc = None
        for t in range(v0 + (1 if r0 else 0), v1):
            acc = cols(d2, t) if acc is None else jnp.minimum(acc, cols(d2, t))
        if r0:
            acc = jnp.minimum(acc, jnp.where(lane >= r0, cols(d2, v0), inf))
        if r1:
            acc = jnp.minimum(acc, jnp.where(lane < r1, cols(d2, v1), inf))
        return jnp.min(acc, axis=1)

    def chunk_scan(d2, lo, hi, bound):
        b = bound[:, None]
        v0, r0 = divmod(lo, 128)
        v1, r1 = divmod(hi, 128)
        acc = None
        for t in range(v0 + (1 if r0 else 0), v1):
            cand = jnp.where(cols(d2, t) < b, cols(iota, t), inf)
            acc = cand if acc is None else jnp.minimum(acc, cand)
        if r0:
            cand = jnp.where((cols(d2, v0) < b) & (lane >= r0),
                             cols(iota, v0), inf)
            acc = jnp.minimum(acc, cand)
        if r1:
            cand = jnp.where((cols(d2, v1) < b) & (lane < r1),
                             cols(iota, v1), inf)
            acc = jnp.minimum(acc, cand)
        return jnp.min(acc, axis=1)

    d2 = None

    def chunk_reduce(lo, hi):
        mraw = chunk_min(d2, lo, hi)
        m = jnp.maximum(mraw, 0.0)
        s = jnp.sqrt(m)
        g = s * 4097.0
        sh = g - (g - s)
        sl = s - sh
        p = s * s
        e = ((sh * sh - p) + 2.0 * (sh * sl)) + sl * sl
        sb = jax.lax.bitcast_convert_type(s, jnp.int32)
        h = jax.lax.bitcast_convert_type(
            sb & jnp.int32(0x7F800000), jnp.float32) * jnp.float32(2.0 ** -23)
        t = e + s * h
        mb = jax.lax.bitcast_convert_type(m, jnp.int32)
        npass = jnp.int32(0)
        for j_ulp in range(1, 4):
            u = jax.lax.bitcast_convert_type(mb + j_ulp, jnp.float32)
            npass = npass + ((u - p) <= t).astype(jnp.int32)
        hi_b = jax.lax.bitcast_convert_type(mb + 1 + npass, jnp.float32)
        idx = chunk_scan(d2, lo, hi, hi_b)
        return s, idx

    s1, i1 = chunk_reduce(0, CHUNK)
    s2, i2 = chunk_reduce(CHUNK, 2 * CHUNK)
    s3, i3 = chunk_reduce(2 * CHUNK, K)

    r = _bf16_round(s1)
    idx = i1
    upd = s2 < r
    idx = jnp.where(upd, i2, idx)
    r = _bf16_round(jnp.where(upd, s2, r))
    upd = s3 < r
    idx = jnp.where(upd, i3, idx)
    out_ref[0, 0, :] = idx.astype(jnp.int32)


def kernel(latents, codebook):
    b, l, d = latents.shape
    n = b * l
    x = latents.reshape(n, d)

    x2 = pl.pallas_call(
        _x2_kernel,
        grid=(n // XB,),
        in_specs=[pl.BlockSpec((XB, d), lambda i: (i, 0))],
        out_specs=pl.BlockSpec((XB, 128), lambda i: (i, 0)),
        out_shape=jax.ShapeDtypeStruct((n, 128), jnp.float32),
    )(x)

    nblk = n // RB
    out = pl.pallas_call(
        _vq_kernel,
        grid=(nblk,),
        in_specs=[
            pl.BlockSpec((RB, d), lambda i: (i, 0)),
            pl.BlockSpec((K, d), lambda i: (0, 0)),
            pl.BlockSpec((RB, 128), lambda i: (i, 0)),
        ],
        out_specs=pl.BlockSpec((1, 1, RB), lambda i: (i, 0, 0)),
        out_shape=jax.ShapeDtypeStruct((nblk, 1, RB), jnp.int32),
        scratch_shapes=[pltpu.VMEM((1, K), jnp.float32)],
    )(x, codebook, x2)
    return out.reshape(b, l)
